# baseline (device time: 29078 ns/iter reference)
import jax
import jax.numpy as jnp
from jax import lax
from jax.experimental import pallas as pl
from jax.experimental.pallas import tpu as pltpu

N_DEV = 4
CHUNKS = 8
LAG = 8


def kernel(x, W1, W2):
    m, _ = x.shape
    n = W2.shape[1]
    mc = m // CHUNKS

    def body(x_ref, w1_ref, w2_ref, out_ref, sbuf, rbuf, send_sems, recv_sems):
        my = lax.axis_index("i")
        p1 = my ^ 1
        p2 = 3 - my
        partner_order = [(p1, p2) if c % 2 == 0 else (p2, p1) for c in range(CHUNKS)]

        barrier_sem = pltpu.get_barrier_semaphore()
        for nbr in [p1, p2]:
            pl.semaphore_signal(
                barrier_sem, inc=1,
                device_id=(nbr,), device_id_type=pl.DeviceIdType.MESH,
            )
        pl.semaphore_wait(barrier_sem, 2)

        w1 = w1_ref[...].astype(jnp.bfloat16)
        w2 = w2_ref[...].astype(jnp.bfloat16)

        def make_rdma(c, stage, partner):
            return pltpu.make_async_remote_copy(
                src_ref=sbuf.at[c, stage],
                dst_ref=rbuf.at[c, stage],
                send_sem=send_sems.at[c, stage],
                recv_sem=recv_sems.at[c, stage],
                device_id=(partner,),
                device_id_type=pl.DeviceIdType.MESH,
            )

        rdma1 = [None] * CHUNKS
        rdma2 = [None] * CHUNKS

        def stage1_done(c):
            rdma1[c].wait_recv()
            s = sbuf[c, 0].astype(jnp.float32) + rbuf[c, 0].astype(jnp.float32)
            sbuf[c, 1] = s.astype(jnp.bfloat16)
            rdma2[c] = make_rdma(c, 1, partner_order[c][1])
            rdma2[c].start()

        for c in range(CHUNKS):
            xc = x_ref[pl.ds(c * mc, mc), :].astype(jnp.bfloat16)
            h = jnp.dot(xc, w1, preferred_element_type=jnp.float32)
            h = jnp.maximum(h, 0.0).astype(jnp.bfloat16)
            sbuf[c, 0] = jnp.dot(
                h, w2, preferred_element_type=jnp.float32
            ).astype(jnp.bfloat16)
            rdma1[c] = make_rdma(c, 0, partner_order[c][0])
            rdma1[c].start()
            if c >= LAG:
                stage1_done(c - LAG)
        for c in range(max(CHUNKS - LAG, 0), CHUNKS):
            stage1_done(c)

        for c in range(CHUNKS):
            rdma2[c].wait_recv()
            out_ref[pl.ds(c * mc, mc), :] = (
                sbuf[c, 1].astype(jnp.float32) + rbuf[c, 1].astype(jnp.float32)
            )

        for c in range(CHUNKS):
            rdma1[c].wait_send()
            rdma2[c].wait_send()

    return pl.pallas_call(
        body,
        out_shape=jax.ShapeDtypeStruct((m, n), jnp.float32),
        in_specs=[pl.BlockSpec(memory_space=pltpu.VMEM)] * 3,
        out_specs=pl.BlockSpec(memory_space=pltpu.VMEM),
        scratch_shapes=[
            pltpu.VMEM((CHUNKS, 2, mc, n), jnp.bfloat16),
            pltpu.VMEM((CHUNKS, 2, mc, n), jnp.bfloat16),
            pltpu.SemaphoreType.DMA((CHUNKS, 2)),
            pltpu.SemaphoreType.DMA((CHUNKS, 2)),
        ],
        compiler_params=pltpu.CompilerParams(collective_id=0),
    )(x, W1, W2)


# device time: 26859 ns/iter; 1.0826x vs baseline; 1.0826x over previous
import jax
import jax.numpy as jnp
from jax import lax
from jax.experimental import pallas as pl
from jax.experimental.pallas import tpu as pltpu

N_DEV = 4
SIZES = [192, 192, 128, 128, 64, 64]
CHUNKS = len(SIZES)
OFFS = [sum(SIZES[:c]) for c in range(CHUNKS)]
MAXC = max(SIZES)


def kernel(x, W1, W2):
    m, _ = x.shape
    n = W2.shape[1]
    assert sum(SIZES) == m

    def body(x_ref, w1_ref, w2_ref, out_ref, sbuf, rbuf, send_sems, recv_sems):
        my = lax.axis_index("i")
        p1 = my ^ 1
        p2 = 3 - my
        partner_order = [(p1, p2) if c % 2 == 0 else (p2, p1) for c in range(CHUNKS)]

        barrier_sem = pltpu.get_barrier_semaphore()
        for nbr in [p1, p2]:
            pl.semaphore_signal(
                barrier_sem, inc=1,
                device_id=(nbr,), device_id_type=pl.DeviceIdType.MESH,
            )

        w1 = w1_ref[...].astype(jnp.bfloat16)
        w2 = w2_ref[...].astype(jnp.bfloat16)

        def make_rdma(c, stage, partner):
            return pltpu.make_async_remote_copy(
                src_ref=sbuf.at[c, stage, pl.ds(0, SIZES[c])],
                dst_ref=rbuf.at[c, stage, pl.ds(0, SIZES[c])],
                send_sem=send_sems.at[c, stage],
                recv_sem=recv_sems.at[c, stage],
                device_id=(partner,),
                device_id_type=pl.DeviceIdType.MESH,
            )

        part = [None] * CHUNKS
        rdma1 = [None] * CHUNKS
        for c in range(CHUNKS):
            xc = x_ref[pl.ds(OFFS[c], SIZES[c]), :].astype(jnp.bfloat16)
            h = jnp.dot(xc, w1, preferred_element_type=jnp.float32)
            h = jnp.maximum(h, 0.0).astype(jnp.bfloat16)
            part[c] = jnp.dot(h, w2, preferred_element_type=jnp.float32)
            sbuf[c, 0, pl.ds(0, SIZES[c])] = part[c].astype(jnp.bfloat16)
            if c == 1:
                pl.semaphore_wait(barrier_sem, 2)
                rdma1[0] = make_rdma(0, 0, partner_order[0][0])
                rdma1[0].start()
            if c >= 1:
                rdma1[c] = make_rdma(c, 0, partner_order[c][0])
                rdma1[c].start()

        rdma2 = [None] * CHUNKS
        for c in range(CHUNKS):
            rdma1[c].wait_recv()
            part[c] = part[c] + rbuf[c, 0, pl.ds(0, SIZES[c])].astype(jnp.float32)
            sbuf[c, 1, pl.ds(0, SIZES[c])] = part[c].astype(jnp.bfloat16)
            rdma2[c] = make_rdma(c, 1, partner_order[c][1])
            rdma2[c].start()

        for c in range(CHUNKS):
            rdma2[c].wait_recv()
            out_ref[pl.ds(OFFS[c], SIZES[c]), :] = (
                part[c] + rbuf[c, 1, pl.ds(0, SIZES[c])].astype(jnp.float32)
            )

        for c in range(CHUNKS):
            rdma1[c].wait_send()
            rdma2[c].wait_send()

    return pl.pallas_call(
        body,
        out_shape=jax.ShapeDtypeStruct((m, n), jnp.float32),
        in_specs=[pl.BlockSpec(memory_space=pltpu.VMEM)] * 3,
        out_specs=pl.BlockSpec(memory_space=pltpu.VMEM),
        scratch_shapes=[
            pltpu.VMEM((CHUNKS, 2, MAXC, n), jnp.bfloat16),
            pltpu.VMEM((CHUNKS, 2, MAXC, n), jnp.bfloat16),
            pltpu.SemaphoreType.DMA((CHUNKS, 2)),
            pltpu.SemaphoreType.DMA((CHUNKS, 2)),
        ],
        compiler_params=pltpu.CompilerParams(collective_id=0),
    )(x, W1, W2)


# device time: 26568 ns/iter; 1.0945x vs baseline; 1.0110x over previous
import jax
import jax.numpy as jnp
from jax import lax
from jax.experimental import pallas as pl
from jax.experimental.pallas import tpu as pltpu

N_DEV = 4
SIZES = [160, 160, 160, 160, 96, 32]
CHUNKS = len(SIZES)
OFFS = [sum(SIZES[:c]) for c in range(CHUNKS)]
MAXC = max(SIZES)


def kernel(x, W1, W2):
    m, _ = x.shape
    n = W2.shape[1]
    assert sum(SIZES) == m

    def body(x_ref, w1_ref, w2_ref, out_ref, sbuf, rbuf, send_sems, recv_sems):
        my = lax.axis_index("i")
        p1 = my ^ 1
        p2 = 3 - my
        partner_order = [(p1, p2) if c % 2 == 0 else (p2, p1) for c in range(CHUNKS)]

        barrier_sem = pltpu.get_barrier_semaphore()
        for nbr in [p1, p2]:
            pl.semaphore_signal(
                barrier_sem, inc=1,
                device_id=(nbr,), device_id_type=pl.DeviceIdType.MESH,
            )

        w1 = w1_ref[...].astype(jnp.bfloat16)
        w2 = w2_ref[...].astype(jnp.bfloat16)

        def make_rdma(c, stage, partner):
            return pltpu.make_async_remote_copy(
                src_ref=sbuf.at[c, stage, pl.ds(0, SIZES[c])],
                dst_ref=rbuf.at[c, stage, pl.ds(0, SIZES[c])],
                send_sem=send_sems.at[c, stage],
                recv_sem=recv_sems.at[c, stage],
                device_id=(partner,),
                device_id_type=pl.DeviceIdType.MESH,
            )

        part = [None] * CHUNKS
        rdma1 = [None] * CHUNKS
        rdma2 = [None] * CHUNKS
        LAG = 4

        def stage1_done(c):
            rdma1[c].wait_recv()
            part[c] = part[c] + rbuf[c, 0, pl.ds(0, SIZES[c])].astype(jnp.float32)
            sbuf[c, 1, pl.ds(0, SIZES[c])] = part[c].astype(jnp.bfloat16)
            rdma2[c] = make_rdma(c, 1, partner_order[c][1])
            rdma2[c].start()

        for c in range(CHUNKS):
            xc = x_ref[pl.ds(OFFS[c], SIZES[c]), :].astype(jnp.bfloat16)
            h = jnp.dot(xc, w1, preferred_element_type=jnp.float32)
            h = jnp.maximum(h, 0.0).astype(jnp.bfloat16)
            part[c] = jnp.dot(h, w2, preferred_element_type=jnp.float32)
            sbuf[c, 0, pl.ds(0, SIZES[c])] = part[c].astype(jnp.bfloat16)
            if c == 1:
                pl.semaphore_wait(barrier_sem, 2)
                rdma1[0] = make_rdma(0, 0, partner_order[0][0])
                rdma1[0].start()
            if c >= 1:
                rdma1[c] = make_rdma(c, 0, partner_order[c][0])
                rdma1[c].start()
            if c >= LAG:
                stage1_done(c - LAG)

        for c in range(max(CHUNKS - LAG, 0), CHUNKS):
            stage1_done(c)

        for c in range(CHUNKS):
            rdma2[c].wait_recv()
            out_ref[pl.ds(OFFS[c], SIZES[c]), :] = (
                part[c] + rbuf[c, 1, pl.ds(0, SIZES[c])].astype(jnp.float32)
            )

        for c in range(CHUNKS):
            rdma1[c].wait_send()
            rdma2[c].wait_send()

    return pl.pallas_call(
        body,
        out_shape=jax.ShapeDtypeStruct((m, n), jnp.float32),
        in_specs=[pl.BlockSpec(memory_space=pltpu.VMEM)] * 3,
        out_specs=pl.BlockSpec(memory_space=pltpu.VMEM),
        scratch_shapes=[
            pltpu.VMEM((CHUNKS, 2, MAXC, n), jnp.bfloat16),
            pltpu.VMEM((CHUNKS, 2, MAXC, n), jnp.bfloat16),
            pltpu.SemaphoreType.DMA((CHUNKS, 2)),
            pltpu.SemaphoreType.DMA((CHUNKS, 2)),
        ],
        compiler_params=pltpu.CompilerParams(collective_id=0),
    )(x, W1, W2)
